# baseline (speedup 1.0000x reference)
import functools

import jax
import jax.numpy as jnp
from jax import lax
from jax.experimental import pallas as pl
from jax.experimental.pallas import tpu as pltpu
from jax.experimental.pallas import tpu_sc as plsc

DIM = 16
BATCH = 16384
ROWS_PER_BLOCK = 8
BLOCK_W = ROWS_PER_BLOCK * DIM
NUM_CORES = 2
NUM_SUBCORES = 16
NUM_WORKERS = NUM_CORES * NUM_SUBCORES
ROWS_PER_WORKER = BATCH // NUM_WORKERS
CHUNK = 128
CHUNKS_PER_WORKER = ROWS_PER_WORKER // CHUNK
GROUPS_PER_CHUNK = CHUNK // 16


def _mfbias_body(ui_hbm, ii_hbm, ub_hbm, ii_blk_hbm, ui_blk_hbm,
                 ut_hbm, it_hbm, ib_hbm, gb_hbm,
                 out_hbm,
                 uidx_v, iidx_v, ubid_v, ibid_v,
                 ublk_v, iblk_v, ub_v, ib_v, gb_v, out_v, sem):
    wid = lax.axis_index("s") * NUM_CORES + lax.axis_index("c")
    crow0 = wid * CHUNKS_PER_WORKER

    pltpu.sync_copy(ui_hbm.at[pl.ds(crow0, CHUNKS_PER_WORKER)], uidx_v)
    pltpu.sync_copy(ii_hbm.at[pl.ds(crow0, CHUNKS_PER_WORKER)], iidx_v)
    pltpu.sync_copy(ui_blk_hbm.at[pl.ds(crow0, CHUNKS_PER_WORKER)], ubid_v)
    pltpu.sync_copy(ii_blk_hbm.at[pl.ds(crow0, CHUNKS_PER_WORKER)], ibid_v)
    pltpu.sync_copy(gb_hbm, gb_v)

    bias_handles = []
    for j in range(CHUNKS_PER_WORKER):
        dst = pl.ds(j * CHUNK, CHUNK)
        bias_handles.append(pltpu.async_copy(
            ub_hbm.at[uidx_v.at[j]], ub_v.at[dst], sem))
        bias_handles.append(pltpu.async_copy(
            ib_hbm.at[iidx_v.at[j]], ib_v.at[dst], sem))

    gb = gb_v[...]
    lane = lax.iota(jnp.int32, 16)

    for j in range(CHUNKS_PER_WORKER):
        hu = pltpu.async_copy(ut_hbm.at[ubid_v.at[j]], ublk_v, sem)
        hi = pltpu.async_copy(it_hbm.at[ibid_v.at[j]], iblk_v, sem)
        hu.wait()
        hi.wait()
        for g in range(GROUPS_PER_CHUNK):
            r0 = g * 16
            rows = jnp.full((16,), r0, jnp.int32) + lane
            ucol0 = (uidx_v[j, pl.ds(r0, 16)] & 7) << 4
            icol0 = (iidx_v[j, pl.ds(r0, 16)] & 7) << 4
            acc = gb
            for d in range(DIM):
                u = plsc.load_gather(ublk_v, [rows, ucol0 + d])
                v = plsc.load_gather(iblk_v, [rows, icol0 + d])
                acc = acc + u * v
            out_v[pl.ds(j * CHUNK + r0, 16)] = acc

    for h in bias_handles:
        h.wait()
    for t in range(ROWS_PER_WORKER // 16):
        s = pl.ds(t * 16, 16)
        out_v[s] = out_v[s] + ub_v[s] + ib_v[s]
    pltpu.sync_copy(out_v, out_hbm.at[pl.ds(wid * ROWS_PER_WORKER,
                                            ROWS_PER_WORKER)])


@functools.partial(jax.jit)
def _mfbias_call(ui2, ii2, user_bias, ii_blk, ui_blk,
                 ut_blocks, it_blocks, item_bias, gb16):
    mesh = plsc.VectorSubcoreMesh(core_axis_name="c", subcore_axis_name="s")
    run = pl.kernel(
        _mfbias_body,
        out_type=jax.ShapeDtypeStruct((BATCH,), jnp.float32),
        mesh=mesh,
        compiler_params=pltpu.CompilerParams(
            needs_layout_passes=False, use_tc_tiling_on_sc=False),
        scratch_types=[
            pltpu.VMEM((CHUNKS_PER_WORKER, CHUNK), jnp.int32),
            pltpu.VMEM((CHUNKS_PER_WORKER, CHUNK), jnp.int32),
            pltpu.VMEM((CHUNKS_PER_WORKER, CHUNK), jnp.int32),
            pltpu.VMEM((CHUNKS_PER_WORKER, CHUNK), jnp.int32),
            pltpu.VMEM((CHUNK, BLOCK_W), jnp.float32),
            pltpu.VMEM((CHUNK, BLOCK_W), jnp.float32),
            pltpu.VMEM((ROWS_PER_WORKER,), jnp.float32),
            pltpu.VMEM((ROWS_PER_WORKER,), jnp.float32),
            pltpu.VMEM((16,), jnp.float32),
            pltpu.VMEM((ROWS_PER_WORKER,), jnp.float32),
            pltpu.SemaphoreType.DMA,
        ],
    )
    return run(ui2, ii2, user_bias, ii_blk, ui_blk,
               ut_blocks, it_blocks, item_bias, gb16)


def kernel(user_indices, item_indices, user_table, item_table, user_bias,
           item_bias, global_bias):
    nrow = NUM_WORKERS * CHUNKS_PER_WORKER
    ui = user_indices.astype(jnp.int32)
    ii = item_indices.astype(jnp.int32)
    ui2 = ui.reshape(nrow, CHUNK)
    ii2 = ii.reshape(nrow, CHUNK)
    ui_blk = (ui >> 3).reshape(nrow, CHUNK)
    ii_blk = (ii >> 3).reshape(nrow, CHUNK)
    ut_blocks = user_table.reshape(-1, BLOCK_W)
    it_blocks = item_table.reshape(-1, BLOCK_W)
    gb16 = jnp.broadcast_to(global_bias.astype(jnp.float32), (16,))
    return _mfbias_call(ui2, ii2, user_bias, ii_blk, ui_blk,
                        ut_blocks, it_blocks, item_bias, gb16)
